# Initial kernel scaffold; baseline (speedup 1.0000x reference)
import jax
import jax.numpy as jnp
from jax.experimental import pallas as pl


def kernel(context, A_tables, C_last):
    raise NotImplementedError("write your pallas kernel here")


# speedup vs baseline: 32.6738x; 32.6738x over previous
import functools

import jax
import jax.numpy as jnp
from jax import lax
from jax.experimental import pallas as pl
from jax.experimental.pallas import tpu as pltpu
from jax.experimental.pallas import tpu_sc as plsc

B, M, S = 1024, 50, 20
EMB = 32
NWORDS = 100000
NSEGS = B * M
NW = 32
SEG_PER_W = NSEGS // NW
NSEG_CHUNK = 64
IDS_CHUNK = NSEG_CHUNK * S
GROWS = 128
NGATHER = IDS_CHUNK // GROWS
NCHUNKS = SEG_PER_W // NSEG_CHUNK
CTX_ROWS_PER_CHUNK = IDS_CHUNK // GROWS


def _lanes_max(x):
    i = lax.iota(jnp.int32, 16)
    for sh in (8, 4, 2, 1):
        x = jnp.maximum(x, x.at[i ^ sh].get(mode="promise_in_bounds"))
    return x


def _lanes_sum(x):
    i = lax.iota(jnp.int32, 16)
    for sh in (8, 4, 2, 1):
        x = x + x.at[i ^ sh].get(mode="promise_in_bounds")
    return x


def _encoder_body(ctx_hbm, at_hbm, cl_hbm, out_hbm,
                  idx_raw, idx0, idx1, idx2, r1, r2, r3, outv, sem):
    cid = lax.axis_index("c")
    sid = lax.axis_index("s")
    wid = sid * 2 + cid

    @pl.loop(0, NCHUNKS)
    def _chunk(c):
        id0 = wid * (SEG_PER_W * S) + c * IDS_CHUNK
        pltpu.sync_copy(ctx_hbm.at[pl.ds(id0, IDS_CHUNK)], idx_raw)
        for r in range(CTX_ROWS_PER_CHUNK):
            for k in range(GROWS // 16):
                v = idx_raw[pl.ds(r * GROWS + k * 16, 16)]
                idx0[r, pl.ds(k * 16, 16)] = v
                idx1[r, pl.ds(k * 16, 16)] = v + NWORDS
                idx2[r, pl.ds(k * 16, 16)] = v + 2 * NWORDS
        cps = []
        for j in range(NGATHER):
            dst = pl.ds(j * GROWS, GROWS)
            cps.append(pltpu.async_copy(at_hbm.at[idx1.at[j]], r1.at[dst], sem))
            cps.append(pltpu.async_copy(at_hbm.at[idx2.at[j]], r2.at[dst], sem))
            cps.append(pltpu.async_copy(cl_hbm.at[idx0.at[j]], r3.at[dst], sem))
        for cp in cps:
            cp.wait()

        @pl.loop(0, NSEG_CHUNK)
        def _seg(s):
            base = s * S

            def red(j, accs):
                a1l, a1h, a2l, a2h, a3l, a3h = accs
                r = base + j
                return (a1l + r1[r, 0:16], a1h + r1[r, 16:32],
                        a2l + r2[r, 0:16], a2h + r2[r, 16:32],
                        a3l + r3[r, 0:16], a3h + r3[r, 16:32])

            z = jnp.zeros((16,), jnp.float32)
            g1l, g1h, g2l, g2h, g3l, g3h = lax.fori_loop(
                0, S, red, (z, z, z, z, z, z))
            o1l = g1l * (1.0 / EMB)
            o1h = g1h * (1.0 / EMB)
            t2l = g1l * o1l
            t2h = g1h * o1h
            m2 = _lanes_max(jnp.maximum(t2l, t2h))
            e2l = jnp.exp(t2l - m2)
            e2h = jnp.exp(t2h - m2)
            inv2 = 1.0 / _lanes_sum(e2l + e2h)
            q2l = o1l + g2l * (e2l * inv2)
            q2h = o1h + g2h * (e2h * inv2)
            t3l = g2l * q2l
            t3h = g2h * q2h
            m3 = _lanes_max(jnp.maximum(t3l, t3h))
            e3l = jnp.exp(t3l - m3)
            e3h = jnp.exp(t3h - m3)
            inv3 = 1.0 / _lanes_sum(e3l + e3h)
            outv[s, 0:16] = g3l * (e3l * inv3)
            outv[s, 16:32] = g3h * (e3h * inv3)

        seg0 = wid * SEG_PER_W + c * NSEG_CHUNK
        pltpu.sync_copy(outv, out_hbm.at[pl.ds(seg0, NSEG_CHUNK)])


@jax.jit
def _encoder_sc(ctx2d, at2, c_last):
    mesh = plsc.VectorSubcoreMesh(core_axis_name="c", subcore_axis_name="s")
    call = pl.kernel(
        _encoder_body,
        out_type=jax.ShapeDtypeStruct((NSEGS, EMB), jnp.float32),
        mesh=mesh,
        scratch_types=[
            pltpu.VMEM((IDS_CHUNK,), jnp.int32),
            pltpu.VMEM((CTX_ROWS_PER_CHUNK, GROWS), jnp.int32),
            pltpu.VMEM((CTX_ROWS_PER_CHUNK, GROWS), jnp.int32),
            pltpu.VMEM((CTX_ROWS_PER_CHUNK, GROWS), jnp.int32),
            pltpu.VMEM((IDS_CHUNK, EMB), jnp.float32),
            pltpu.VMEM((IDS_CHUNK, EMB), jnp.float32),
            pltpu.VMEM((IDS_CHUNK, EMB), jnp.float32),
            pltpu.VMEM((NSEG_CHUNK, EMB), jnp.float32),
            pltpu.SemaphoreType.DMA,
        ],
        compiler_params=pltpu.CompilerParams(use_tc_tiling_on_sc=False),
    )
    return call(ctx2d, at2, c_last)


def kernel(context, A_tables, C_last):
    ctx2d = context.reshape(-1)
    at2 = A_tables.reshape(3 * NWORDS, EMB)
    out = _encoder_sc(ctx2d, at2, C_last)
    return out.reshape(B, M, EMB)
